# Initial kernel scaffold; baseline (speedup 1.0000x reference)
import jax
import jax.numpy as jnp
from jax.experimental import pallas as pl


def kernel(input1, input2, input3, W1, W2, W3, W4, g1, b1, g2, b2, g3, b3, g4, b4):
    raise NotImplementedError("write your pallas kernel here")


# speedup vs baseline: 60.6376x; 60.6376x over previous
import functools

import jax
import jax.numpy as jnp
from jax.experimental import pallas as pl


def _conv_kern(x_ref, w_ref, m_ref, y_ref, s1_ref, s2_ref, *, wp):
    x = x_ref[0]
    acc = None
    for t in range(9):
        dy, dx = t // 3, t % 3
        o = (dy - 1) * wp + (dx - 1)
        xs = x if o == 0 else jnp.roll(x, -o, axis=0)
        c = jnp.dot(xs, w_ref[t], preferred_element_type=jnp.float32)
        acc = c if acc is None else acc + c
    acc = acc * m_ref[...]
    y_ref[0] = acc
    s1_ref[0] = jnp.sum(acc, axis=0, keepdims=True)
    s2_ref[0] = jnp.sum(acc * acc, axis=0, keepdims=True)


def _conv_layer(x, w, wp):
    b, p, cin = x.shape
    mask = jnp.pad(jnp.ones((wp - 2, wp - 2), jnp.float32), 1).reshape(p, 1)
    kern = functools.partial(_conv_kern, wp=wp)
    return pl.pallas_call(
        kern,
        grid=(b,),
        in_specs=[
            pl.BlockSpec((1, p, cin), lambda i: (i, 0, 0)),
            pl.BlockSpec((9, cin, 64), lambda i: (0, 0, 0)),
            pl.BlockSpec((p, 1), lambda i: (0, 0)),
        ],
        out_specs=[
            pl.BlockSpec((1, p, 64), lambda i: (i, 0, 0)),
            pl.BlockSpec((1, 1, 64), lambda i: (i, 0, 0)),
            pl.BlockSpec((1, 1, 64), lambda i: (i, 0, 0)),
        ],
        out_shape=[
            jax.ShapeDtypeStruct((b, p, 64), jnp.float32),
            jax.ShapeDtypeStruct((b, 1, 64), jnp.float32),
            jax.ShapeDtypeStruct((b, 1, 64), jnp.float32),
        ],
    )(x, w, mask)


def _apply_kern(y_ref, sc_ref, sh_ref, m_ref, o_ref, *, normalize):
    v = y_ref[0] * sc_ref[0] + sh_ref[0]
    v = jnp.where(v >= 0, v, 0.2 * v)
    v = v * m_ref[...]
    if normalize:
        ss = jnp.sum(v * v, axis=1, keepdims=True)
        v = v * jax.lax.rsqrt(jnp.where(ss > 0, ss, 1.0))
    o_ref[0] = v


def _apply_layer(y, scale, shift, wp, normalize=False):
    b, p, c = y.shape
    mask = jnp.pad(jnp.ones((wp - 2, wp - 2), jnp.float32), 1).reshape(p, 1)
    kern = functools.partial(_apply_kern, normalize=normalize)
    return pl.pallas_call(
        kern,
        grid=(b,),
        in_specs=[
            pl.BlockSpec((1, p, c), lambda i: (i, 0, 0)),
            pl.BlockSpec((1, 1, c), lambda i: (i, 0, 0)),
            pl.BlockSpec((1, 1, c), lambda i: (i, 0, 0)),
            pl.BlockSpec((p, 1), lambda i: (0, 0)),
        ],
        out_specs=pl.BlockSpec((1, p, c), lambda i: (i, 0, 0)),
        out_shape=jax.ShapeDtypeStruct((b, p, c), jnp.float32),
    )(y, scale.reshape(b, 1, c), shift.reshape(b, 1, c), mask)


_NEG = -4.0


def _row_top3(x):
    r, m = x.shape
    chunk = 2304
    if m > chunk:
        cands = []
        for s in range(0, m, chunk):
            e = min(s + chunk, m)
            cands.append(_row_top3(x[:, s:e]))
        return _row_top3(jnp.concatenate(cands, axis=1))
    iota = jax.lax.broadcasted_iota(jnp.int32, x.shape, 1)
    outs = []
    for t in range(3):
        mx = jnp.max(x, axis=1, keepdims=True)
        outs.append(mx)
        if t < 2:
            key = jnp.where(x >= mx, iota, jnp.int32(2 ** 30))
            j = jnp.min(key, axis=1, keepdims=True)
            x = jnp.where(iota == j, _NEG, x)
    return jnp.concatenate(outs, axis=1)


_DN = (((1,), (1,)), ((), ()))


def _sim_kern(q_ref, s_ref, o_ref):
    q = q_ref[0]
    ip = jax.lax.dot_general(q, s_ref[0], _DN,
                             preferred_element_type=jnp.float32)
    t3 = _row_top3(ip)
    o_ref[0, 0] = jnp.sum(t3).reshape(1, 1)


def _sim_call(q, s):
    bq = q.shape[0]
    nc, m, c = s.shape
    out = pl.pallas_call(
        _sim_kern,
        grid=(bq, nc),
        in_specs=[
            pl.BlockSpec((1, q.shape[1], c), lambda i, j: (i, 0, 0)),
            pl.BlockSpec((1, m, c), lambda i, j: (j, 0, 0)),
        ],
        out_specs=pl.BlockSpec((1, 1, 1, 1), lambda i, j: (i, j, 0, 0)),
        out_shape=jax.ShapeDtypeStruct((bq, nc, 1, 1), jnp.float32),
    )(q, s)
    return out.reshape(bq, nc)


def _final_kern(q_ref, s_ref, u_ref, cm_ref, o_ref):
    q = q_ref[0]
    ips = jax.lax.dot_general(q, s_ref[0], _DN,
                              preferred_element_type=jnp.float32)
    ipu = jax.lax.dot_general(q, u_ref[...], _DN,
                              preferred_element_type=jnp.float32)
    ipu = ipu + cm_ref[0]
    cand = jnp.concatenate([_row_top3(ips), _row_top3(ipu)], axis=1)
    t3 = _row_top3(cand)
    o_ref[0, 0] = jnp.sum(t3).reshape(1, 1)


def _final_call(q, s, u, cmask):
    bq = q.shape[0]
    nc, m, c = s.shape
    mu = u.shape[0]
    out = pl.pallas_call(
        _final_kern,
        grid=(bq, nc),
        in_specs=[
            pl.BlockSpec((1, q.shape[1], c), lambda i, j: (i, 0, 0)),
            pl.BlockSpec((1, m, c), lambda i, j: (j, 0, 0)),
            pl.BlockSpec((mu, c), lambda i, j: (0, 0)),
            pl.BlockSpec((1, 1, mu), lambda i, j: (j, 0, 0)),
        ],
        out_specs=pl.BlockSpec((1, 1, 1, 1), lambda i, j: (i, j, 0, 0)),
        out_shape=jax.ShapeDtypeStruct((bq, nc, 1, 1), jnp.float32),
    )(q, s, u, cmask)
    return out.reshape(bq, nc)


def _pool_and_pad(z, wp):
    b = z.shape[0]
    hv = wp - 2
    z = z.reshape(b, wp, wp, 64)[:, 1:1 + hv, 1:1 + hv, :]
    z = z.reshape(b, hv // 2, 2, hv // 2, 2, 64).max(axis=(2, 4))
    z = jnp.pad(z, ((0, 0), (1, 1), (1, 1), (0, 0)))
    return z.reshape(b, (hv // 2 + 2) ** 2, 64)


def _bn_scale_shift(s1, s2, seg_ids, seg_sizes, nvalid, gamma, beta,
                    eps=1e-5):
    s1 = s1.reshape(s1.shape[0], 64)
    s2 = s2.reshape(s2.shape[0], 64)
    seg1 = jax.ops.segment_sum(s1, seg_ids, num_segments=7)
    seg2 = jax.ops.segment_sum(s2, seg_ids, num_segments=7)
    cnt = (seg_sizes * nvalid).astype(jnp.float32).reshape(7, 1)
    mean = seg1 / cnt
    var = seg2 / cnt - mean * mean
    scale = gamma.reshape(1, 64) * jax.lax.rsqrt(var + eps)
    shift = beta.reshape(1, 64) - mean * scale
    return scale[seg_ids], shift[seg_ids]


def kernel(input1, input2, input3, W1, W2, W3, W4,
           g1, b1, g2, b2, g3, b3, g4, b4):
    x = jnp.concatenate([input1, input2.reshape(25, 3, 84, 84), input3])
    x = jnp.pad(x.transpose(0, 2, 3, 1), ((0, 0), (1, 1), (1, 1), (0, 0)))
    x = x.reshape(65, 86 * 86, 3)
    seg_ids7 = jnp.array([0] * 25 + [1] * 5 + [2] * 5 + [3] * 5 + [4] * 5
                         + [5] * 5 + [6] * 15, dtype=jnp.int32)
    seg_sizes = jnp.array([25, 5, 5, 5, 5, 5, 15], dtype=jnp.float32)

    def conv_bn(x, w, g, b, wp, normalize=False):
        wr = w.transpose(2, 3, 1, 0).reshape(9, w.shape[1], 64)
        y, s1, s2 = _conv_layer(x, wr, wp)
        nvalid = (wp - 2) ** 2
        sc, sh = _bn_scale_shift(s1, s2, seg_ids7, seg_sizes, nvalid, g, b)
        return _apply_layer(y, sc, sh, wp, normalize=normalize)

    z = conv_bn(x, W1, g1, b1, 86)
    z = _pool_and_pad(z, 86)
    z = conv_bn(z, W2, g2, b2, 44)
    z = _pool_and_pad(z, 44)
    z = conv_bn(z, W3, g3, b3, 23)
    z = conv_bn(z, W4, g4, b4, 23, normalize=True)

    d = z.reshape(65, 23, 23, 64)[:, 1:22, 1:22, :].reshape(65, 441, 64)
    q_n = d[:25]
    s_n = d[25:50].reshape(5, 5 * 441, 64)
    u_n = d[50:]

    sim_u = _sim_call(u_n, s_n)
    sim_u = jax.nn.softmax(sim_u, axis=1)
    _, sel = jax.lax.top_k(sim_u.T, 10)
    selmask = jnp.zeros((5, 15), jnp.float32)
    selmask = selmask.at[jnp.arange(5)[:, None], sel].set(1.0)
    cmask = (selmask - 1.0) * 4.0
    cmask = jnp.repeat(cmask, 441, axis=1).reshape(5, 1, 15 * 441)

    u_flat = u_n.reshape(15 * 441, 64)
    return _final_call(q_n, s_n, u_flat, cmask)


# speedup vs baseline: 87.5283x; 1.4435x over previous
import functools

import jax
import jax.numpy as jnp
from jax.experimental import pallas as pl


def _conv_kern(x_ref, w_ref, m_ref, y_ref, s1_ref, s2_ref, *, wp):
    x = x_ref[0]
    acc = None
    for t in range(9):
        dy, dx = t // 3, t % 3
        o = (dy - 1) * wp + (dx - 1)
        xs = x if o == 0 else jnp.roll(x, -o, axis=0)
        c = jnp.dot(xs, w_ref[t], preferred_element_type=jnp.float32)
        acc = c if acc is None else acc + c
    acc = acc * m_ref[...]
    y_ref[0] = acc
    s1_ref[0] = jnp.sum(acc, axis=0, keepdims=True)
    s2_ref[0] = jnp.sum(acc * acc, axis=0, keepdims=True)


def _conv_layer(x, w, wp):
    b, p, cin = x.shape
    mask = jnp.pad(jnp.ones((wp - 2, wp - 2), jnp.float32), 1).reshape(p, 1)
    kern = functools.partial(_conv_kern, wp=wp)
    return pl.pallas_call(
        kern,
        grid=(b,),
        in_specs=[
            pl.BlockSpec((1, p, cin), lambda i: (i, 0, 0)),
            pl.BlockSpec((9, cin, 64), lambda i: (0, 0, 0)),
            pl.BlockSpec((p, 1), lambda i: (0, 0)),
        ],
        out_specs=[
            pl.BlockSpec((1, p, 64), lambda i: (i, 0, 0)),
            pl.BlockSpec((1, 1, 64), lambda i: (i, 0, 0)),
            pl.BlockSpec((1, 1, 64), lambda i: (i, 0, 0)),
        ],
        out_shape=[
            jax.ShapeDtypeStruct((b, p, 64), jnp.float32),
            jax.ShapeDtypeStruct((b, 1, 64), jnp.float32),
            jax.ShapeDtypeStruct((b, 1, 64), jnp.float32),
        ],
    )(x, w, mask)


def _apply_kern(y_ref, sc_ref, sh_ref, m_ref, o_ref, *, normalize):
    v = y_ref[0] * sc_ref[0] + sh_ref[0]
    v = jnp.where(v >= 0, v, 0.2 * v)
    if m_ref is not None:
        v = v * m_ref[...]
    if normalize:
        ss = jnp.sum(v * v, axis=1, keepdims=True)
        v = v * jax.lax.rsqrt(jnp.where(ss > 0, ss, 1.0))
    o_ref[0] = v


def _apply_layer(y, scale, shift, wp=None, normalize=False):
    b, p, c = y.shape
    if wp is not None:
        mask = jnp.pad(jnp.ones((wp - 2, wp - 2), jnp.float32),
                       1).reshape(p, 1)
        args = (y, scale.reshape(b, 1, c), shift.reshape(b, 1, c), mask)
        mspec = [pl.BlockSpec((p, 1), lambda i: (0, 0))]
        kern = functools.partial(_apply_kern, normalize=normalize)
    else:
        args = (y, scale.reshape(b, 1, c), shift.reshape(b, 1, c))
        mspec = []
        kern = functools.partial(
            lambda y_ref, sc_ref, sh_ref, o_ref, normalize: _apply_kern(
                y_ref, sc_ref, sh_ref, None, o_ref, normalize=normalize),
            normalize=normalize)
    return pl.pallas_call(
        kern,
        grid=(b,),
        in_specs=[
            pl.BlockSpec((1, p, c), lambda i: (i, 0, 0)),
            pl.BlockSpec((1, 1, c), lambda i: (i, 0, 0)),
            pl.BlockSpec((1, 1, c), lambda i: (i, 0, 0)),
        ] + mspec,
        out_specs=pl.BlockSpec((1, p, c), lambda i: (i, 0, 0)),
        out_shape=jax.ShapeDtypeStruct((b, p, c), jnp.float32),
    )(*args)


_NEG = -4.0

_DN = (((1,), (1,)), ((), ()))


def _row_top3(x):
    outs = []
    for t in range(3):
        m = jnp.max(x, axis=1, keepdims=True)
        outs.append(m)
        if t < 2:
            x = jnp.where(x >= m, _NEG, x)
    return jnp.concatenate(outs, axis=1)


def _sim_kern(q_ref, s_ref, o_ref):
    q = q_ref[0]
    outs = []
    for j in range(5):
        ip = jax.lax.dot_general(q, s_ref[j], _DN,
                                 preferred_element_type=jnp.float32)
        t3 = _row_top3(ip)
        outs.append(jnp.sum(t3).reshape(1, 1))
    o_ref[0] = jnp.concatenate(outs, axis=1)


def _sim_call(q, s):
    bq = q.shape[0]
    nc, m, c = s.shape
    out = pl.pallas_call(
        _sim_kern,
        grid=(bq,),
        in_specs=[
            pl.BlockSpec((1, q.shape[1], c), lambda i: (i, 0, 0)),
            pl.BlockSpec((nc, m, c), lambda i: (0, 0, 0)),
        ],
        out_specs=pl.BlockSpec((1, 1, nc), lambda i: (i, 0, 0)),
        out_shape=jax.ShapeDtypeStruct((bq, 1, nc), jnp.float32),
    )(q, s)
    return out.reshape(bq, nc)


def _final_kern(q_ref, s_ref, u_ref, cm_ref, o_ref):
    q = q_ref[0]
    cu = []
    for t in range(15):
        ip = jax.lax.dot_general(q, u_ref[t], _DN,
                                 preferred_element_type=jnp.float32)
        cu.append(_row_top3(ip))
    cand_u = jnp.concatenate(cu, axis=1)
    outs = []
    for j in range(5):
        ips = jax.lax.dot_general(q, s_ref[j], _DN,
                                  preferred_element_type=jnp.float32)
        t3s = _row_top3(ips)
        cand = jnp.concatenate([t3s, cand_u + cm_ref[j]], axis=1)
        t3 = _row_top3(cand)
        outs.append(jnp.sum(t3).reshape(1, 1))
    o_ref[0] = jnp.concatenate(outs, axis=1)


def _final_call(q, s, u, cmask):
    bq = q.shape[0]
    nc, m, c = s.shape
    out = pl.pallas_call(
        _final_kern,
        grid=(bq,),
        in_specs=[
            pl.BlockSpec((1, q.shape[1], c), lambda i: (i, 0, 0)),
            pl.BlockSpec((nc, m, c), lambda i: (0, 0, 0)),
            pl.BlockSpec(u.shape, lambda i: (0, 0, 0)),
            pl.BlockSpec(cmask.shape, lambda i: (0, 0, 0)),
        ],
        out_specs=pl.BlockSpec((1, 1, nc), lambda i: (i, 0, 0)),
        out_shape=jax.ShapeDtypeStruct((bq, 1, nc), jnp.float32),
    )(q, s, u, cmask)
    return out.reshape(bq, nc)


def _pool_and_pad(z, hv):
    b = z.shape[0]
    z = z.reshape(b, hv // 2, 2, hv // 2, 2, 64).max(axis=(2, 4))
    z = jnp.pad(z, ((0, 0), (1, 1), (1, 1), (0, 0)))
    return z.reshape(b, (hv // 2 + 2) ** 2, 64)


def _bn_scale_shift(s1, s2, seg_ids, seg_sizes, nvalid, gamma, beta,
                    eps=1e-5):
    s1 = s1.reshape(s1.shape[0], 64)
    s2 = s2.reshape(s2.shape[0], 64)
    seg1 = jax.ops.segment_sum(s1, seg_ids, num_segments=7)
    seg2 = jax.ops.segment_sum(s2, seg_ids, num_segments=7)
    cnt = (seg_sizes * nvalid).astype(jnp.float32).reshape(7, 1)
    mean = seg1 / cnt
    var = seg2 / cnt - mean * mean
    scale = gamma.reshape(1, 64) * jax.lax.rsqrt(var + eps)
    shift = beta.reshape(1, 64) - mean * scale
    return scale[seg_ids], shift[seg_ids]


def kernel(input1, input2, input3, W1, W2, W3, W4,
           g1, b1, g2, b2, g3, b3, g4, b4):
    seg_ids7 = jnp.array([0] * 25 + [1] * 5 + [2] * 5 + [3] * 5 + [4] * 5
                         + [5] * 5 + [6] * 15, dtype=jnp.int32)
    seg_sizes = jnp.array([25, 5, 5, 5, 5, 5, 15], dtype=jnp.float32)

    def bn(s1, s2, nvalid, g, b):
        return _bn_scale_shift(s1, s2, seg_ids7, seg_sizes, nvalid, g, b)

    x = jnp.concatenate([input1, input2.reshape(25, 3, 84, 84), input3])
    x = jnp.pad(x.transpose(0, 2, 3, 1), ((0, 0), (1, 1), (1, 1), (0, 0)))
    x = x.reshape(65, 86 * 86, 3)
    w1 = W1.transpose(2, 3, 1, 0).reshape(9, 3, 64)
    y, s1, s2 = _conv_layer(x, w1, 86)
    sc, sh = bn(s1, s2, 84 * 84, g1, b1)
    z = _apply_layer(y, sc, sh, wp=86)
    z = _pool_and_pad(z.reshape(65, 86, 86, 64)[:, 1:85, 1:85], 84)

    def conv_bn(z, w, g, b, wp, normalize=False):
        wr = w.transpose(2, 3, 1, 0).reshape(9, 64, 64)
        y, s1, s2 = _conv_layer(z, wr, wp)
        sc, sh = bn(s1, s2, (wp - 2) ** 2, g, b)
        return _apply_layer(y, sc, sh, wp=wp, normalize=normalize)

    z = conv_bn(z, W2, g2, b2, 44)
    z = _pool_and_pad(z.reshape(65, 44, 44, 64)[:, 1:43, 1:43], 42)
    z = conv_bn(z, W3, g3, b3, 23)
    z = conv_bn(z, W4, g4, b4, 23, normalize=True)

    d = z.reshape(65, 23, 23, 64)[:, 1:22, 1:22, :].reshape(65, 441, 64)
    q_n = d[:25]
    s_n = d[25:50].reshape(5, 5 * 441, 64)
    u_n = d[50:]

    sim_u = _sim_call(u_n, s_n)
    sim_u = jax.nn.softmax(sim_u, axis=1)
    _, sel = jax.lax.top_k(sim_u.T, 10)
    selmask = jnp.sum(jax.nn.one_hot(sel, 15, dtype=jnp.float32), axis=1)
    cmask = (selmask - 1.0) * 4.0
    cmask = jnp.broadcast_to(cmask[:, :, None], (5, 15, 3)).reshape(5, 45)
    cmask = cmask.reshape(5, 1, 45)

    return _final_call(q_n, s_n, u_n, cmask)


# speedup vs baseline: 144.7142x; 1.6533x over previous
import functools

import jax
import jax.numpy as jnp
from jax.experimental import pallas as pl
from jax.experimental.pallas import tpu as pltpu


def _conv_kern(x_ref, w_ref, m_ref, y_ref, s1_ref, s2_ref, *, wp):
    x = x_ref[0]
    acc = None
    for t in range(9):
        dy, dx = t // 3, t % 3
        o = (dy - 1) * wp + (dx - 1)
        xs = x if o == 0 else jnp.roll(x, -o, axis=0)
        c = jnp.dot(xs, w_ref[t], preferred_element_type=jnp.float32)
        acc = c if acc is None else acc + c
    acc = acc * m_ref[...]
    y_ref[0] = acc
    s1_ref[0] = jnp.sum(acc, axis=0, keepdims=True)
    s2_ref[0] = jnp.sum(acc * acc, axis=0, keepdims=True)


def _conv_layer(x, w, wp):
    b, p, cin = x.shape
    mask = jnp.pad(jnp.ones((wp - 2, wp - 2), jnp.float32), 1).reshape(p, 1)
    kern = functools.partial(_conv_kern, wp=wp)
    return pl.pallas_call(
        kern,
        grid=(b,),
        in_specs=[
            pl.BlockSpec((1, p, cin), lambda i: (i, 0, 0)),
            pl.BlockSpec((9, cin, 64), lambda i: (0, 0, 0)),
            pl.BlockSpec((p, 1), lambda i: (0, 0)),
        ],
        out_specs=[
            pl.BlockSpec((1, p, 64), lambda i: (i, 0, 0)),
            pl.BlockSpec((1, 1, 64), lambda i: (i, 0, 0)),
            pl.BlockSpec((1, 1, 64), lambda i: (i, 0, 0)),
        ],
        out_shape=[
            jax.ShapeDtypeStruct((b, p, 64), jnp.float32),
            jax.ShapeDtypeStruct((b, 1, 64), jnp.float32),
            jax.ShapeDtypeStruct((b, 1, 64), jnp.float32),
        ],
    )(x, w, mask)


def _apply_kern(y_ref, sc_ref, sh_ref, m_ref, o_ref, *, normalize):
    v = y_ref[0] * sc_ref[0] + sh_ref[0]
    v = jnp.where(v >= 0, v, 0.2 * v)
    if m_ref is not None:
        v = v * m_ref[...]
    if normalize:
        ss = jnp.sum(v * v, axis=1, keepdims=True)
        v = v * jax.lax.rsqrt(jnp.where(ss > 0, ss, 1.0))
    o_ref[0] = v


def _apply_layer(y, scale, shift, wp=None, normalize=False):
    b, p, c = y.shape
    if wp is not None:
        mask = jnp.pad(jnp.ones((wp - 2, wp - 2), jnp.float32),
                       1).reshape(p, 1)
        args = (y, scale.reshape(b, 1, c), shift.reshape(b, 1, c), mask)
        mspec = [pl.BlockSpec((p, 1), lambda i: (0, 0))]
        kern = functools.partial(_apply_kern, normalize=normalize)
    else:
        args = (y, scale.reshape(b, 1, c), shift.reshape(b, 1, c))
        mspec = []
        kern = functools.partial(
            lambda y_ref, sc_ref, sh_ref, o_ref, normalize: _apply_kern(
                y_ref, sc_ref, sh_ref, None, o_ref, normalize=normalize),
            normalize=normalize)
    return pl.pallas_call(
        kern,
        grid=(b,),
        in_specs=[
            pl.BlockSpec((1, p, c), lambda i: (i, 0, 0)),
            pl.BlockSpec((1, 1, c), lambda i: (i, 0, 0)),
            pl.BlockSpec((1, 1, c), lambda i: (i, 0, 0)),
        ] + mspec,
        out_specs=pl.BlockSpec((1, p, c), lambda i: (i, 0, 0)),
        out_shape=jax.ShapeDtypeStruct((b, p, c), jnp.float32),
    )(*args)


def _apply_pool_kern(y_ref, sc_ref, sh_ref, o_ref, rm_ref, *, wp):
    v = y_ref[0] * sc_ref[0] + sh_ref[0]
    v = jnp.where(v >= 0, v, 0.2 * v)
    cm = jnp.maximum(v, jnp.roll(v, -1, axis=0))
    rm_ref[...] = jnp.maximum(cm, jnp.roll(cm, -wp, axis=0))
    nh = (wp - 2) // 2
    wp2 = nh + 2
    o_ref[0] = jnp.zeros((wp2 * wp2, 64), jnp.float32)
    for h in range(nh):
        src = rm_ref[pl.Slice(wp * (2 * h + 1) + 1, nh, 2), :]
        o_ref[0, pl.ds(wp2 * (h + 1) + 1, nh), :] = src


def _apply_pool_layer(y, scale, shift, wp):
    b, p, c = y.shape
    nh = (wp - 2) // 2
    p2 = (nh + 2) ** 2
    kern = functools.partial(_apply_pool_kern, wp=wp)
    return pl.pallas_call(
        kern,
        grid=(b,),
        in_specs=[
            pl.BlockSpec((1, p, c), lambda i: (i, 0, 0)),
            pl.BlockSpec((1, 1, c), lambda i: (i, 0, 0)),
            pl.BlockSpec((1, 1, c), lambda i: (i, 0, 0)),
        ],
        out_specs=pl.BlockSpec((1, p2, c), lambda i: (i, 0, 0)),
        out_shape=jax.ShapeDtypeStruct((b, p2, c), jnp.float32),
        scratch_shapes=[pltpu.VMEM((p, c), jnp.float32)],
    )(y, scale.reshape(b, 1, c), shift.reshape(b, 1, c))


_NEG = -4.0

_DN = (((1,), (1,)), ((), ()))


def _row_top3(x):
    outs = []
    for t in range(3):
        m = jnp.max(x, axis=1, keepdims=True)
        outs.append(m)
        if t < 2:
            x = jnp.where(x >= m, _NEG, x)
    return jnp.concatenate(outs, axis=1)


def _sim_kern(q_ref, s_ref, o_ref):
    q = q_ref[0]
    outs = []
    for j in range(5):
        ip = jax.lax.dot_general(q, s_ref[j], _DN,
                                 preferred_element_type=jnp.float32)
        t3 = _row_top3(ip)
        outs.append(jnp.sum(t3).reshape(1, 1))
    o_ref[0] = jnp.concatenate(outs, axis=1)


def _sim_call(q, s):
    bq = q.shape[0]
    nc, m, c = s.shape
    out = pl.pallas_call(
        _sim_kern,
        grid=(bq,),
        in_specs=[
            pl.BlockSpec((1, q.shape[1], c), lambda i: (i, 0, 0)),
            pl.BlockSpec((nc, m, c), lambda i: (0, 0, 0)),
        ],
        out_specs=pl.BlockSpec((1, 1, nc), lambda i: (i, 0, 0)),
        out_shape=jax.ShapeDtypeStruct((bq, 1, nc), jnp.float32),
    )(q, s)
    return out.reshape(bq, nc)


def _final_kern(q_ref, s_ref, u_ref, cm_ref, o_ref):
    q = q_ref[0]
    cu = []
    for t in range(15):
        ip = jax.lax.dot_general(q, u_ref[t], _DN,
                                 preferred_element_type=jnp.float32)
        cu.append(_row_top3(ip))
    cand_u = jnp.concatenate(cu, axis=1)
    outs = []
    for j in range(5):
        ips = jax.lax.dot_general(q, s_ref[j], _DN,
                                  preferred_element_type=jnp.float32)
        t3s = _row_top3(ips)
        cand = jnp.concatenate([t3s, cand_u + cm_ref[j]], axis=1)
        t3 = _row_top3(cand)
        outs.append(jnp.sum(t3).reshape(1, 1))
    o_ref[0] = jnp.concatenate(outs, axis=1)


def _final_call(q, s, u, cmask):
    bq = q.shape[0]
    nc, m, c = s.shape
    out = pl.pallas_call(
        _final_kern,
        grid=(bq,),
        in_specs=[
            pl.BlockSpec((1, q.shape[1], c), lambda i: (i, 0, 0)),
            pl.BlockSpec((nc, m, c), lambda i: (0, 0, 0)),
            pl.BlockSpec(u.shape, lambda i: (0, 0, 0)),
            pl.BlockSpec(cmask.shape, lambda i: (0, 0, 0)),
        ],
        out_specs=pl.BlockSpec((1, 1, nc), lambda i: (i, 0, 0)),
        out_shape=jax.ShapeDtypeStruct((bq, 1, nc), jnp.float32),
    )(q, s, u, cmask)
    return out.reshape(bq, nc)


def _bn_scale_shift(s1, s2, seg_oh, seg_sizes, nvalid, gamma, beta,
                    eps=1e-5):
    s1 = s1.reshape(s1.shape[0], 64)
    s2 = s2.reshape(s2.shape[0], 64)
    seg1 = seg_oh @ s1
    seg2 = seg_oh @ s2
    cnt = (seg_sizes * nvalid).astype(jnp.float32).reshape(7, 1)
    mean = seg1 / cnt
    var = seg2 / cnt - mean * mean
    scale = gamma.reshape(1, 64) * jax.lax.rsqrt(var + eps)
    shift = beta.reshape(1, 64) - mean * scale
    return seg_oh.T @ scale, seg_oh.T @ shift


def kernel(input1, input2, input3, W1, W2, W3, W4,
           g1, b1, g2, b2, g3, b3, g4, b4):
    seg_ids7 = jnp.array([0] * 25 + [1] * 5 + [2] * 5 + [3] * 5 + [4] * 5
                         + [5] * 5 + [6] * 15, dtype=jnp.int32)
    seg_oh = (seg_ids7[None, :] == jnp.arange(7)[:, None]).astype(jnp.float32)
    seg_sizes = jnp.array([25, 5, 5, 5, 5, 5, 15], dtype=jnp.float32)

    def bn(s1, s2, nvalid, g, b):
        return _bn_scale_shift(s1, s2, seg_oh, seg_sizes, nvalid, g, b)

    x = jnp.concatenate([input1, input2.reshape(25, 3, 84, 84), input3])
    x = jnp.pad(x.transpose(0, 2, 3, 1), ((0, 0), (1, 1), (1, 1), (0, 0)))
    x = x.reshape(65, 86 * 86, 3)
    w1 = W1.transpose(2, 3, 1, 0).reshape(9, 3, 64)
    y, s1, s2 = _conv_layer(x, w1, 86)
    sc, sh = bn(s1, s2, 84 * 84, g1, b1)
    z = _apply_pool_layer(y, sc, sh, 86)

    def conv_bn(z, w, g, b, wp, pool=False, normalize=False):
        wr = w.transpose(2, 3, 1, 0).reshape(9, 64, 64)
        y, s1, s2 = _conv_layer(z, wr, wp)
        sc, sh = bn(s1, s2, (wp - 2) ** 2, g, b)
        if pool:
            return _apply_pool_layer(y, sc, sh, wp)
        return _apply_layer(y, sc, sh, wp=wp, normalize=normalize)

    z = conv_bn(z, W2, g2, b2, 44, pool=True)
    z = conv_bn(z, W3, g3, b3, 23)
    z = conv_bn(z, W4, g4, b4, 23, normalize=True)

    d = z.reshape(65, 23, 23, 64)[:, 1:22, 1:22, :].reshape(65, 441, 64)
    q_n = d[:25]
    s_n = d[25:50].reshape(5, 5 * 441, 64)
    u_n = d[50:]

    sim_u = _sim_call(u_n, s_n)
    sim_u = jax.nn.softmax(sim_u, axis=1)
    _, sel = jax.lax.top_k(sim_u.T, 10)
    selmask = jnp.sum(jax.nn.one_hot(sel, 15, dtype=jnp.float32), axis=1)
    cmask = (selmask - 1.0) * 4.0
    cmask = jnp.broadcast_to(cmask[:, :, None], (5, 15, 3)).reshape(5, 45)
    cmask = cmask.reshape(5, 1, 45)

    return _final_call(q_n, s_n, u_n, cmask)


# speedup vs baseline: 152.9660x; 1.0570x over previous
import functools

import jax
import jax.numpy as jnp
from jax.experimental import pallas as pl
from jax.experimental.pallas import tpu as pltpu


def _conv_kern(x_ref, w_ref, m_ref, y_ref, s1_ref, s2_ref, *, wp):
    x = x_ref[0]
    acc = None
    for t in range(9):
        dy, dx = t // 3, t % 3
        o = (dy - 1) * wp + (dx - 1)
        xs = x if o == 0 else jnp.roll(x, -o, axis=0)
        c = jnp.dot(xs, w_ref[t], preferred_element_type=jnp.float32)
        acc = c if acc is None else acc + c
    acc = acc * m_ref[...]
    y_ref[0] = acc
    s1_ref[0] = jnp.sum(acc, axis=0, keepdims=True)
    s2_ref[0] = jnp.sum(acc * acc, axis=0, keepdims=True)


def _conv_layer(x, w, wp):
    b, p, cin = x.shape
    mask = jnp.pad(jnp.ones((wp - 2, wp - 2), jnp.float32), 1).reshape(p, 1)
    kern = functools.partial(_conv_kern, wp=wp)
    return pl.pallas_call(
        kern,
        grid=(b,),
        in_specs=[
            pl.BlockSpec((1, p, cin), lambda i: (i, 0, 0)),
            pl.BlockSpec((9, cin, 64), lambda i: (0, 0, 0)),
            pl.BlockSpec((p, 1), lambda i: (0, 0)),
        ],
        out_specs=[
            pl.BlockSpec((1, p, 64), lambda i: (i, 0, 0)),
            pl.BlockSpec((1, 1, 64), lambda i: (i, 0, 0)),
            pl.BlockSpec((1, 1, 64), lambda i: (i, 0, 0)),
        ],
        out_shape=[
            jax.ShapeDtypeStruct((b, p, 64), jnp.float32),
            jax.ShapeDtypeStruct((b, 1, 64), jnp.float32),
            jax.ShapeDtypeStruct((b, 1, 64), jnp.float32),
        ],
    )(x, w, mask)


def _apply_kern(y_ref, sc_ref, sh_ref, m_ref, o_ref, *, normalize):
    v = y_ref[0] * sc_ref[0] + sh_ref[0]
    v = jnp.where(v >= 0, v, 0.2 * v)
    if m_ref is not None:
        v = v * m_ref[...]
    if normalize:
        ss = jnp.sum(v * v, axis=1, keepdims=True)
        v = v * jax.lax.rsqrt(jnp.where(ss > 0, ss, 1.0))
    o_ref[0] = v


def _apply_norm_desc_kern(y_ref, sc_ref, sh_ref, o_ref, *, wp):
    v = y_ref[0] * sc_ref[0] + sh_ref[0]
    v = jnp.where(v >= 0, v, 0.2 * v)
    ss = jnp.sum(v * v, axis=1, keepdims=True)
    v = v * jax.lax.rsqrt(jnp.where(ss > 0, ss, 1.0))
    hv = wp - 2
    for h in range(hv):
        o_ref[0, pl.ds(h * hv, hv), :] = v[wp * (h + 1) + 1:
                                           wp * (h + 1) + 1 + hv]


def _apply_norm_desc_layer(y, scale, shift, wp):
    b, p, c = y.shape
    hv = wp - 2
    kern = functools.partial(_apply_norm_desc_kern, wp=wp)
    return pl.pallas_call(
        kern,
        grid=(b,),
        in_specs=[
            pl.BlockSpec((1, p, c), lambda i: (i, 0, 0)),
            pl.BlockSpec((1, 1, c), lambda i: (i, 0, 0)),
            pl.BlockSpec((1, 1, c), lambda i: (i, 0, 0)),
        ],
        out_specs=pl.BlockSpec((1, hv * hv, c), lambda i: (i, 0, 0)),
        out_shape=jax.ShapeDtypeStruct((b, hv * hv, c), jnp.float32),
    )(y, scale.reshape(b, 1, c), shift.reshape(b, 1, c))


def _apply_layer(y, scale, shift, wp=None, normalize=False):
    b, p, c = y.shape
    if wp is not None:
        mask = jnp.pad(jnp.ones((wp - 2, wp - 2), jnp.float32),
                       1).reshape(p, 1)
        args = (y, scale.reshape(b, 1, c), shift.reshape(b, 1, c), mask)
        mspec = [pl.BlockSpec((p, 1), lambda i: (0, 0))]
        kern = functools.partial(_apply_kern, normalize=normalize)
    else:
        args = (y, scale.reshape(b, 1, c), shift.reshape(b, 1, c))
        mspec = []
        kern = functools.partial(
            lambda y_ref, sc_ref, sh_ref, o_ref, normalize: _apply_kern(
                y_ref, sc_ref, sh_ref, None, o_ref, normalize=normalize),
            normalize=normalize)
    return pl.pallas_call(
        kern,
        grid=(b,),
        in_specs=[
            pl.BlockSpec((1, p, c), lambda i: (i, 0, 0)),
            pl.BlockSpec((1, 1, c), lambda i: (i, 0, 0)),
            pl.BlockSpec((1, 1, c), lambda i: (i, 0, 0)),
        ] + mspec,
        out_specs=pl.BlockSpec((1, p, c), lambda i: (i, 0, 0)),
        out_shape=jax.ShapeDtypeStruct((b, p, c), jnp.float32),
    )(*args)


def _apply_pool_kern(y_ref, sc_ref, sh_ref, o_ref, rm_ref, *, wp):
    v = y_ref[0] * sc_ref[0] + sh_ref[0]
    v = jnp.where(v >= 0, v, 0.2 * v)
    cm = jnp.maximum(v, jnp.roll(v, -1, axis=0))
    rm_ref[...] = jnp.maximum(cm, jnp.roll(cm, -wp, axis=0))
    nh = (wp - 2) // 2
    wp2 = nh + 2
    o_ref[0] = jnp.zeros((wp2 * wp2, 64), jnp.float32)
    for h in range(nh):
        src = rm_ref[pl.Slice(wp * (2 * h + 1) + 1, nh, 2), :]
        o_ref[0, pl.ds(wp2 * (h + 1) + 1, nh), :] = src


def _apply_pool_layer(y, scale, shift, wp):
    b, p, c = y.shape
    nh = (wp - 2) // 2
    p2 = (nh + 2) ** 2
    kern = functools.partial(_apply_pool_kern, wp=wp)
    return pl.pallas_call(
        kern,
        grid=(b,),
        in_specs=[
            pl.BlockSpec((1, p, c), lambda i: (i, 0, 0)),
            pl.BlockSpec((1, 1, c), lambda i: (i, 0, 0)),
            pl.BlockSpec((1, 1, c), lambda i: (i, 0, 0)),
        ],
        out_specs=pl.BlockSpec((1, p2, c), lambda i: (i, 0, 0)),
        out_shape=jax.ShapeDtypeStruct((b, p2, c), jnp.float32),
        scratch_shapes=[pltpu.VMEM((p, c), jnp.float32)],
    )(y, scale.reshape(b, 1, c), shift.reshape(b, 1, c))


_NEG = -4.0

_DN = (((1,), (1,)), ((), ()))


def _row_top3(x):
    outs = []
    for t in range(3):
        m = jnp.max(x, axis=1, keepdims=True)
        outs.append(m)
        if t < 2:
            x = jnp.where(x >= m, _NEG, x)
    return jnp.concatenate(outs, axis=1)


def _sim_kern(q_ref, s_ref, o_ref):
    q = q_ref[0]
    outs = []
    for j in range(5):
        ip = jax.lax.dot_general(q, s_ref[j], _DN,
                                 preferred_element_type=jnp.float32)
        t3 = _row_top3(ip)
        outs.append(jnp.sum(t3).reshape(1, 1))
    o_ref[0] = jnp.concatenate(outs, axis=1)


def _sim_call(q, s):
    bq = q.shape[0]
    nc, m, c = s.shape
    out = pl.pallas_call(
        _sim_kern,
        grid=(bq,),
        in_specs=[
            pl.BlockSpec((1, q.shape[1], c), lambda i: (i, 0, 0)),
            pl.BlockSpec((nc, m, c), lambda i: (0, 0, 0)),
        ],
        out_specs=pl.BlockSpec((1, 1, nc), lambda i: (i, 0, 0)),
        out_shape=jax.ShapeDtypeStruct((bq, 1, nc), jnp.float32),
    )(q, s)
    return out.reshape(bq, nc)


def _final_kern(q_ref, s_ref, u_ref, cm_ref, o_ref):
    q = q_ref[0]
    cu = []
    for t in range(15):
        ip = jax.lax.dot_general(q, u_ref[t], _DN,
                                 preferred_element_type=jnp.float32)
        cu.append(_row_top3(ip))
    cand_u = jnp.concatenate(cu, axis=1)
    outs = []
    for j in range(5):
        ips = jax.lax.dot_general(q, s_ref[j], _DN,
                                  preferred_element_type=jnp.float32)
        t3s = _row_top3(ips)
        cand = jnp.concatenate([t3s, cand_u + cm_ref[j]], axis=1)
        t3 = _row_top3(cand)
        outs.append(jnp.sum(t3).reshape(1, 1))
    o_ref[0] = jnp.concatenate(outs, axis=1)


def _final_call(q, s, u, cmask):
    bq = q.shape[0]
    nc, m, c = s.shape
    out = pl.pallas_call(
        _final_kern,
        grid=(bq,),
        in_specs=[
            pl.BlockSpec((1, q.shape[1], c), lambda i: (i, 0, 0)),
            pl.BlockSpec((nc, m, c), lambda i: (0, 0, 0)),
            pl.BlockSpec(u.shape, lambda i: (0, 0, 0)),
            pl.BlockSpec(cmask.shape, lambda i: (0, 0, 0)),
        ],
        out_specs=pl.BlockSpec((1, 1, nc), lambda i: (i, 0, 0)),
        out_shape=jax.ShapeDtypeStruct((bq, 1, nc), jnp.float32),
    )(q, s, u, cmask)
    return out.reshape(bq, nc)


def _bn_scale_shift(s1, s2, seg_oh, seg_sizes, nvalid, gamma, beta,
                    eps=1e-5):
    s1 = s1.reshape(s1.shape[0], 64)
    s2 = s2.reshape(s2.shape[0], 64)
    seg1 = seg_oh @ s1
    seg2 = seg_oh @ s2
    cnt = (seg_sizes * nvalid).astype(jnp.float32).reshape(7, 1)
    mean = seg1 / cnt
    var = seg2 / cnt - mean * mean
    scale = gamma.reshape(1, 64) * jax.lax.rsqrt(var + eps)
    shift = beta.reshape(1, 64) - mean * scale
    return seg_oh.T @ scale, seg_oh.T @ shift


def kernel(input1, input2, input3, W1, W2, W3, W4,
           g1, b1, g2, b2, g3, b3, g4, b4):
    seg_ids7 = jnp.array([0] * 25 + [1] * 5 + [2] * 5 + [3] * 5 + [4] * 5
                         + [5] * 5 + [6] * 15, dtype=jnp.int32)
    seg_oh = (seg_ids7[None, :] == jnp.arange(7)[:, None]).astype(jnp.float32)
    seg_sizes = jnp.array([25, 5, 5, 5, 5, 5, 15], dtype=jnp.float32)

    def bn(s1, s2, nvalid, g, b):
        return _bn_scale_shift(s1, s2, seg_oh, seg_sizes, nvalid, g, b)

    x = jnp.concatenate([input1, input2.reshape(25, 3, 84, 84), input3])
    x = jnp.pad(x.transpose(0, 2, 3, 1), ((0, 0), (1, 1), (1, 1), (0, 0)))
    x = x.reshape(65, 86 * 86, 3)
    w1 = W1.transpose(2, 3, 1, 0).reshape(9, 3, 64)
    y, s1, s2 = _conv_layer(x, w1, 86)
    sc, sh = bn(s1, s2, 84 * 84, g1, b1)
    z = _apply_pool_layer(y, sc, sh, 86)

    def conv_bn(z, w, g, b, wp, pool=False, normalize=False):
        wr = w.transpose(2, 3, 1, 0).reshape(9, 64, 64)
        y, s1, s2 = _conv_layer(z, wr, wp)
        sc, sh = bn(s1, s2, (wp - 2) ** 2, g, b)
        if pool:
            return _apply_pool_layer(y, sc, sh, wp)
        if normalize:
            return _apply_norm_desc_layer(y, sc, sh, wp)
        return _apply_layer(y, sc, sh, wp=wp)

    z = conv_bn(z, W2, g2, b2, 44, pool=True)
    z = conv_bn(z, W3, g3, b3, 23)
    d = conv_bn(z, W4, g4, b4, 23, normalize=True)

    q_n = d[:25]
    s_n = d[25:50].reshape(5, 5 * 441, 64)
    u_n = d[50:]

    sim_u = _sim_call(u_n, s_n)
    sim_u = jax.nn.softmax(sim_u, axis=1)
    _, sel = jax.lax.top_k(sim_u.T, 10)
    selmask = jnp.sum(jax.nn.one_hot(sel, 15, dtype=jnp.float32), axis=1)
    cmask = (selmask - 1.0) * 4.0
    cmask = jnp.broadcast_to(cmask[:, :, None], (5, 15, 3)).reshape(5, 45)
    cmask = cmask.reshape(5, 1, 45)

    return _final_call(q_n, s_n, u_n, cmask)
